# baseline (speedup 1.0000x reference)
import functools
import jax
import jax.numpy as jnp
from jax import lax
from jax.experimental import pallas as pl
from jax.experimental.pallas import tpu as pltpu
from jax.experimental.pallas import tpu_sc as plsc

N_NODES = 50000
N_EDGES = 800000
E_PAD = 819200
N_GRAPHS = 64
NUM_TILES = 32


def _make_gather(width, n_sub):
    blk = 128 * n_sub
    per_tile = E_PAD // NUM_TILES
    n_iter = per_tile // blk
    mesh = plsc.VectorSubcoreMesh(core_axis_name="c", subcore_axis_name="s")
    scratch = ([pltpu.VMEM((128,), jnp.int32) for _ in range(n_sub)]
               + [pltpu.VMEM((128, width), jnp.float32) for _ in range(n_sub)]
               + [pltpu.SemaphoreType.DMA])

    @functools.partial(
        pl.kernel, mesh=mesh,
        out_type=jax.ShapeDtypeStruct((E_PAD, width), jnp.float32),
        scratch_types=scratch)
    def k(table, idx, out, *scr):
        idx_bufs = scr[:n_sub]
        row_bufs = scr[n_sub:2 * n_sub]
        sem = scr[-1]
        wid = lax.axis_index("s") * 2 + lax.axis_index("c")
        base0 = wid * per_tile

        def body(i, carry):
            e0 = pl.multiple_of(base0 + i * blk, 8)
            for j in range(n_sub):
                pltpu.sync_copy(idx.at[pl.ds(e0 + j * 128, 128)], idx_bufs[j])
            waits = [pltpu.async_copy(table.at[idx_bufs[j]], row_bufs[j], sem)
                     for j in range(n_sub)]
            for w in waits:
                w.wait()
            for j in range(n_sub):
                pltpu.sync_copy(row_bufs[j], out.at[pl.ds(e0 + j * 128, 128)])
            return carry

        lax.fori_loop(0, n_iter, body, 0)

    return k


def _make_scatter(width):
    n_chunks = width // 16
    per_core = n_chunks // 2
    per_tile = E_PAD // 16
    n_sub = 4
    blk = 128 * n_sub
    n_iter = per_tile // blk
    rows_a = 3128
    rows_last = N_NODES - 15 * rows_a
    mesh = plsc.VectorSubcoreMesh(core_axis_name="c", subcore_axis_name="s")
    scratch = ([pltpu.VMEM((128,), jnp.int32) for _ in range(n_sub)]
               + [pltpu.VMEM((128, 16), jnp.float32) for _ in range(n_sub)]
               + [pltpu.VMEM_SHARED((N_NODES, 16), jnp.float32)])

    @functools.partial(
        pl.kernel, mesh=mesh,
        out_type=jax.ShapeDtypeStruct((n_chunks * N_NODES, 16), jnp.float32),
        scratch_types=scratch)
    def k(p_mat, dst, zeros, out, *scr):
        idx_bufs = scr[:n_sub]
        p_bufs = scr[n_sub:2 * n_sub]
        acc = scr[-1]
        core = lax.axis_index("c")
        sid = lax.axis_index("s")

        row0 = pl.multiple_of(sid * rows_a, 8)

        for j in range(per_core):
            c = 2 * j + core
            ob = pl.multiple_of(c * N_NODES + sid * rows_a, 8)

            @pl.when(sid < 15)
            def _():
                pltpu.sync_copy(zeros.at[pl.ds(row0, rows_a)],
                                acc.at[pl.ds(row0, rows_a)])

            @pl.when(sid == 15)
            def _():
                pltpu.sync_copy(zeros.at[pl.ds(row0, rows_last)],
                                acc.at[pl.ds(row0, rows_last)])

            plsc.subcore_barrier()

            def body(i, carry):
                e0 = pl.multiple_of(sid * per_tile + i * blk, 8)
                for t in range(n_sub):
                    pltpu.sync_copy(dst.at[pl.ds(e0 + t * 128, 128)],
                                    idx_bufs[t])
                    pltpu.sync_copy(
                        p_mat.at[c, pl.ds(e0 + t * 128, 128), :],
                        p_bufs[t])
                for t in range(n_sub):
                    pltpu.sync_copy(p_bufs[t], acc.at[idx_bufs[t]], add=True)
                return carry

            lax.fori_loop(0, n_iter, body, 0)
            plsc.subcore_barrier()

            @pl.when(sid < 15)
            def _():
                pltpu.sync_copy(acc.at[pl.ds(row0, rows_a)],
                                out.at[pl.ds(ob, rows_a)])

            @pl.when(sid == 15)
            def _():
                pltpu.sync_copy(acc.at[pl.ds(row0, rows_last)],
                                out.at[pl.ds(ob, rows_last)])

    return k


def _mm(a, w, bias, block):
    m, kk = a.shape
    nn = w.shape[1]
    assert m % block == 0

    def body(a_ref, w_ref, b_ref, o_ref):
        o_ref[...] = (jnp.dot(a_ref[...], w_ref[...],
                              preferred_element_type=jnp.float32)
                      + b_ref[...])

    return pl.pallas_call(
        body, grid=(m // block,),
        in_specs=[pl.BlockSpec((block, kk), lambda i: (i, 0)),
                  pl.BlockSpec((kk, nn), lambda i: (0, 0)),
                  pl.BlockSpec((1, nn), lambda i: (0, 0))],
        out_specs=pl.BlockSpec((block, nn), lambda i: (i, 0)),
        out_shape=jax.ShapeDtypeStruct((m, nn), jnp.float32),
    )(a, w, bias.reshape(1, nn))


def _edge_weights(xls, xrd, ea, att, w_col):
    width = xls.shape[1]
    blk = 2048

    def body(l_ref, r_ref, e_ref, a_ref, o_ref):
        m = l_ref[...] + r_ref[...] + e_ref[...]
        lk = jnp.where(m >= 0, m, 0.2 * m)
        logit = jnp.dot(lk, a_ref[...],
                        preferred_element_type=jnp.float32)
        w = jnp.exp(logit)
        eid = (pl.program_id(0) * blk
               + lax.broadcasted_iota(jnp.int32, (blk, 1), 0))
        w = jnp.where(eid < N_EDGES, w, 0.0)
        lane = lax.broadcasted_iota(jnp.int32, (blk, width), 1)
        pmat = jnp.where(lane == w_col, w, w * l_ref[...])
        for c in range(width // 16):
            o_ref[c, :, :] = pmat[:, c * 16:(c + 1) * 16]

    n_chunks = width // 16
    return pl.pallas_call(
        body, grid=(E_PAD // blk,),
        in_specs=[pl.BlockSpec((blk, width), lambda i: (i, 0)),
                  pl.BlockSpec((blk, width), lambda i: (i, 0)),
                  pl.BlockSpec((blk, width), lambda i: (i, 0)),
                  pl.BlockSpec((width, 1), lambda i: (0, 0))],
        out_specs=pl.BlockSpec((n_chunks, blk, 16), lambda i: (0, i, 0)),
        out_shape=jax.ShapeDtypeStruct((n_chunks, E_PAD, 16), jnp.float32),
    )(xls, xrd, ea, att.reshape(width, 1))


def _node_finish(acc_t, bias, w_col, d_valid):
    width = acc_t.shape[1]
    blk = 2000

    def body(a_ref, b_ref, o_ref):
        a = a_ref[...]
        den = a[:, w_col:w_col + 1]
        h = jnp.maximum(a / (den + 1e-16) + b_ref[...], 0.0)
        lane = lax.broadcasted_iota(jnp.int32, (blk, width), 1)
        o_ref[...] = jnp.where(lane < d_valid, h, 0.0)

    return pl.pallas_call(
        body, grid=(N_NODES // blk,),
        in_specs=[pl.BlockSpec((blk, width), lambda i: (i, 0)),
                  pl.BlockSpec((1, width), lambda i: (0, 0))],
        out_specs=pl.BlockSpec((blk, width), lambda i: (i, 0)),
        out_shape=jax.ShapeDtypeStruct((N_NODES, width), jnp.float32),
    )(acc_t, bias.reshape(1, width))


def _pool(h, batch3):
    width = h.shape[1]
    blk = 400

    def body(h_ref, b_ref, s_ref, c_ref):
        @pl.when(pl.program_id(0) == 0)
        def _():
            s_ref[...] = jnp.zeros_like(s_ref)
            c_ref[...] = jnp.zeros_like(c_ref)

        bb = b_ref[...].reshape(blk, 1)
        gid = lax.broadcasted_iota(jnp.int32, (blk, N_GRAPHS), 1)
        onehot = (bb == gid).astype(jnp.float32)
        s_ref[...] += lax.dot_general(
            onehot, h_ref[...], dimension_numbers=(((0,), (0,)), ((), ())),
            preferred_element_type=jnp.float32,
            precision=lax.Precision.HIGHEST)
        c_ref[...] += jnp.sum(onehot, axis=0, keepdims=True)

    return pl.pallas_call(
        body, grid=(N_NODES // blk,),
        in_specs=[pl.BlockSpec((blk, width), lambda i: (i, 0)),
                  pl.BlockSpec((1, 1, blk), lambda i: (i, 0, 0))],
        out_specs=[pl.BlockSpec((N_GRAPHS, width), lambda i: (0, 0)),
                   pl.BlockSpec((1, N_GRAPHS), lambda i: (0, 0))],
        out_shape=[jax.ShapeDtypeStruct((N_GRAPHS, width), jnp.float32),
                   jax.ShapeDtypeStruct((1, N_GRAPHS), jnp.float32)],
    )(h, batch3)


def _bn(h, gamma, beta, eps=1e-5):
    mu = jnp.mean(h, axis=0)
    var = jnp.var(h, axis=0)
    return gamma * (h - mu) * lax.rsqrt(var + eps) + beta


def _tail_kernel(g_ref, g4, be4, W5, b5, W6, b6, g6, be6, W7, b7, o_ref):
    g = g_ref[...]
    g = _bn(g, g4[...], be4[...])
    g = jnp.maximum(jnp.dot(g, W5[...], preferred_element_type=jnp.float32)
                    + b5[...], 0.0)
    g = jnp.maximum(jnp.dot(g, W6[...], preferred_element_type=jnp.float32)
                    + b6[...], 0.0)
    g = _bn(g, g6[...], be6[...])
    o_ref[...] = (jnp.dot(g, W7[...], preferred_element_type=jnp.float32)
                  + b7[...])


def _pad2(w, rows, cols):
    return jnp.pad(w, ((0, rows - w.shape[0]), (0, cols - w.shape[1])))


def _bn_affine(h, gamma, beta, eps=1e-5):
    mu = jnp.mean(h, axis=0)
    var = jnp.var(h, axis=0)
    a = gamma * lax.rsqrt(var + eps)
    return a, beta - mu * a


_gather128 = _make_gather(128, 4)
_gather256 = _make_gather(256, 2)
_scatter128 = _make_scatter(128)
_scatter256 = _make_scatter(256)


def kernel(x, edge_attr, params, edge_index, batch):
    p = params
    src = edge_index[0]
    dst = edge_index[1]
    pad_e = E_PAD - N_EDGES
    src_p = jnp.concatenate([src, jnp.zeros((pad_e,), jnp.int32)])
    dst_p = jnp.concatenate([dst, jnp.zeros((pad_e,), jnp.int32)])
    ea_p = jnp.pad(edge_attr, ((0, pad_e), (0, 0)))
    zeros16 = jnp.zeros((N_NODES, 16), jnp.float32)

    xl1 = _mm(x, _pad2(p['Wl1'], 16, 128), jnp.zeros((128,)), 2000)
    xr1 = _mm(x, _pad2(p['Wr1'], 16, 128), jnp.zeros((128,)), 2000)
    ea1 = _mm(ea_p, _pad2(p['We1'], 18, 128), jnp.zeros((128,)), 2048)
    xls1 = _gather128(xl1, src_p)
    xrd1 = _gather128(xr1, dst_p)
    att1 = jnp.pad(p['att1'], (0, 28))
    p1 = _edge_weights(xls1, xrd1, ea1, att1, 100)
    acc1t = jax.ops.segment_sum(
        jnp.swapaxes(p1, 0, 1).reshape(E_PAD, 128), dst_p,
        num_segments=N_NODES)
    h1 = _node_finish(acc1t, jnp.pad(p['b1'], (0, 28)), 100, 100)

    a1, s1 = _bn_affine(h1[:, :100], p['g1'], p['be1'])
    h1bn = jnp.pad(h1[:, :100] * a1 + s1, ((0, 0), (0, 28)))

    xl2 = _mm(h1bn, _pad2(p['Wl2'], 128, 256), jnp.zeros((256,)), 2000)
    xr2 = _mm(h1bn, _pad2(p['Wr2'], 128, 256), jnp.zeros((256,)), 2000)
    ea2 = _mm(ea_p, _pad2(p['We2'], 18, 256), jnp.zeros((256,)), 2048)
    xls2 = _gather256(xl2, src_p)
    xrd2 = _gather256(xr2, dst_p)
    att2 = jnp.pad(p['att2'], (0, 56))
    p2 = _edge_weights(xls2, xrd2, ea2, att2, 200)
    acc2t = jax.ops.segment_sum(
        jnp.swapaxes(p2, 0, 1).reshape(E_PAD, 256), dst_p,
        num_segments=N_NODES)
    h2 = _node_finish(acc2t, jnp.pad(p['b2'], (0, 56)), 200, 200)

    a2, s2 = _bn_affine(h2[:, :200], p['g2'], p['be2'])
    h2bn = jnp.pad(h2[:, :200] * a2 + s2, ((0, 0), (0, 56)))
    h3 = _mm(h2bn, _pad2(p['W3'], 256, 400), p['b3'], 2000)

    a3, s3 = _bn_affine(h3, p['g3'], p['be3'])
    batch3 = batch.reshape(N_NODES // 400, 1, 400)
    s, c = _pool(h3, batch3)
    g = s / jnp.maximum(c.reshape(N_GRAPHS, 1), 1.0)
    g = g * a3 + s3

    out = pl.pallas_call(
        _tail_kernel,
        out_shape=jax.ShapeDtypeStruct((N_GRAPHS, 100), jnp.float32),
    )(g, p['g4'], p['be4'], p['W5'], p['b5'], p['W6'], p['b6'],
      p['g6'], p['be6'], p['W7'], p['b7'])
    return out
